# baseline (speedup 1.0000x reference)
import functools

import jax
import jax.numpy as jnp
from jax import lax
from jax.experimental import pallas as pl
from jax.experimental.pallas import tpu as pltpu
from jax.experimental.pallas import tpu_sc as plsc

_BATCH = 16384
_DIM = 64
_NUM_CORES = 2
_NUM_SUBCORES = 16
_NUM_WORKERS = _NUM_CORES * _NUM_SUBCORES
_B_PER_W = _BATCH // _NUM_WORKERS

_mesh = plsc.VectorSubcoreMesh(core_axis_name="c", subcore_axis_name="s")


@functools.partial(
    pl.kernel,
    mesh=_mesh,
    compiler_params=pltpu.CompilerParams(use_tc_tiling_on_sc=False),
    out_type=jax.ShapeDtypeStruct((_BATCH, _DIM), jnp.float32),
    scratch_types=[
        pltpu.VMEM((_B_PER_W,), jnp.int32),
        pltpu.VMEM((_B_PER_W, _DIM), jnp.float32),
        pltpu.SemaphoreType.DMA,
    ],
)
def _embed_lookup(labels_hbm, table_hbm, out_hbm, idx_v, rows_v, sem):
    wid = lax.axis_index("s") * _NUM_CORES + lax.axis_index("c")
    base = wid * _B_PER_W
    pltpu.sync_copy(labels_hbm.at[pl.ds(base, _B_PER_W)], idx_v)
    pltpu.async_copy(table_hbm.at[idx_v], rows_v, sem).wait()
    pltpu.sync_copy(rows_v, out_hbm.at[pl.ds(base, _B_PER_W)])


def kernel(labels, table):
    return _embed_lookup(labels.astype(jnp.int32), table)


# speedup vs baseline: 1.7178x; 1.7178x over previous
import functools

import jax
import jax.numpy as jnp
from jax import lax
from jax.experimental import pallas as pl
from jax.experimental.pallas import tpu as pltpu
from jax.experimental.pallas import tpu_sc as plsc

_BATCH = 16384
_DIM = 64
_NUM_CORES = 2
_NUM_SUBCORES = 16
_NUM_WORKERS = _NUM_CORES * _NUM_SUBCORES
_B_PER_W = _BATCH // _NUM_WORKERS

_mesh = plsc.VectorSubcoreMesh(core_axis_name="c", subcore_axis_name="s")


@functools.partial(
    pl.kernel,
    mesh=_mesh,
    out_type=jax.ShapeDtypeStruct((_BATCH, _DIM), jnp.float32),
    scratch_types=[
        pltpu.VMEM((_B_PER_W,), jnp.int32),
        pltpu.VMEM((_B_PER_W, _DIM), jnp.float32),
        pltpu.SemaphoreType.DMA,
    ],
)
def _embed_lookup(labels_hbm, table_hbm, out_hbm, lbl_v, rows_v, sem):
    wid = lax.axis_index("s") * _NUM_CORES + lax.axis_index("c")
    base = wid * _B_PER_W
    pltpu.sync_copy(labels_hbm.at[pl.ds(base, _B_PER_W)], lbl_v)

    def fire(ci, carry):
        cbase = ci * 16
        lv = lbl_v[pl.ds(cbase, 16)]
        for j in range(16):
            row = lv[j]
            pltpu.make_async_copy(
                table_hbm.at[pl.ds(row, 1), :],
                rows_v.at[pl.ds(cbase + j, 1), :],
                sem,
            ).start()
        return carry

    lax.fori_loop(0, _B_PER_W // 16, fire, 0)

    def drain(i, carry):
        pltpu.make_async_copy(
            table_hbm.at[pl.ds(0, 1), :],
            rows_v.at[pl.ds(0, 1), :],
            sem,
        ).wait()
        return carry

    lax.fori_loop(0, _B_PER_W, drain, 0, unroll=4)
    pltpu.sync_copy(rows_v, out_hbm.at[pl.ds(base, _B_PER_W)])


def kernel(labels, table):
    return _embed_lookup(labels.astype(jnp.int32), table)


# speedup vs baseline: 1.7266x; 1.0051x over previous
import functools

import jax
import jax.numpy as jnp
from jax import lax
from jax.experimental import pallas as pl
from jax.experimental.pallas import tpu as pltpu
from jax.experimental.pallas import tpu_sc as plsc

_BATCH = 16384
_DIM = 64
_NUM_CORES = 2
_NUM_SUBCORES = 16
_NUM_WORKERS = _NUM_CORES * _NUM_SUBCORES
_B_PER_W = _BATCH // _NUM_WORKERS

_mesh = plsc.VectorSubcoreMesh(core_axis_name="c", subcore_axis_name="s")


@functools.partial(
    pl.kernel,
    mesh=_mesh,
    out_type=jax.ShapeDtypeStruct((_BATCH, _DIM), jnp.float32),
    scratch_types=[
        pltpu.VMEM((_B_PER_W,), jnp.int32),
        pltpu.VMEM((_B_PER_W, _DIM), jnp.float32),
    ] + [pltpu.SemaphoreType.DMA] * 8,
)
def _embed_lookup(labels_hbm, table_hbm, out_hbm, lbl_v, rows_v, *sems):
    wid = lax.axis_index("s") * _NUM_CORES + lax.axis_index("c")
    base = wid * _B_PER_W
    pltpu.sync_copy(labels_hbm.at[pl.ds(base, _B_PER_W)], lbl_v)

    def fire(ci, carry):
        cbase = ci * 16
        lv = lbl_v[pl.ds(cbase, 16)]
        for j in range(16):
            row = lv[j]
            pltpu.make_async_copy(
                table_hbm.at[pl.ds(row, 1), :],
                rows_v.at[pl.ds(cbase + j, 1), :],
                sems[j % 8],
            ).start()
        return carry

    lax.fori_loop(0, _B_PER_W // 16, fire, 0)

    def drain(i, carry):
        for j in range(8):
            pltpu.make_async_copy(
                table_hbm.at[pl.ds(0, 1), :],
                rows_v.at[pl.ds(0, 1), :],
                sems[j],
            ).wait()
        return carry

    lax.fori_loop(0, _B_PER_W // 8, drain, 0)
    pltpu.sync_copy(rows_v, out_hbm.at[pl.ds(base, _B_PER_W)])


def kernel(labels, table):
    return _embed_lookup(labels.astype(jnp.int32), table)
